# Initial kernel scaffold; baseline (speedup 1.0000x reference)
import jax
import jax.numpy as jnp
from jax.experimental import pallas as pl


def kernel(input_ids, logits):
    raise NotImplementedError("write your pallas kernel here")


# speedup vs baseline: 53.0777x; 53.0777x over previous
import functools

import jax
import jax.numpy as jnp
from jax import lax
from jax.experimental import pallas as pl
from jax.experimental.pallas import tpu as pltpu
import numpy as np

_TEMP = np.float32(0.7)
_TOP_P = np.float32(0.9)
_TOP_K = 50
_R = 128
_V = 100000
_RB = 8
_NEG_INF = np.float32(-np.inf)

_KS0 = np.int32(0)
_KS1 = np.int32(42)
_KS2 = np.int32((0 ^ 42 ^ 0x1BD11BDA) - (1 << 32) if (0 ^ 42 ^ 0x1BD11BDA) >= (1 << 31) else (0 ^ 42 ^ 0x1BD11BDA))
_ROTS = (13, 15, 26, 6, 17, 29, 16, 24, 13, 15, 26, 6, 17, 29, 16, 24, 13, 15, 26, 6)
_TINY = np.float32(np.finfo(np.float32).tiny)


def _stage1_body(x_ref, topv_ref, topi_ref, z_ref, scratch_ref):
    l = x_ref[...] / _TEMP
    scratch_ref[...] = l
    m0 = jnp.max(l, axis=-1, keepdims=True)
    z = jnp.sum(jnp.exp(l - m0), axis=-1, keepdims=True)
    z_ref[...] = jnp.broadcast_to(z, (_RB, 128))

    col = lax.broadcasted_iota(jnp.int32, (_RB, _V), 1)
    kcol = lax.broadcasted_iota(jnp.int32, (_RB, 128), 1)

    def step(k, carry):
        tv, tix = carry
        cur = scratch_ref[...]
        m = jnp.max(cur, axis=-1, keepdims=True)
        idx = jnp.min(jnp.where(cur == m, col, jnp.int32(2 ** 30)),
                      axis=-1, keepdims=True)
        kmask = kcol == k
        tv = jnp.where(kmask, m, tv)
        tix = jnp.where(kmask, idx, tix)
        scratch_ref[...] = jnp.where(col == idx, _NEG_INF, cur)
        return tv, tix

    tv0 = jnp.full((_RB, 128), _NEG_INF, jnp.float32)
    ti0 = jnp.zeros((_RB, 128), jnp.int32)
    tv, tix = lax.fori_loop(0, _TOP_K, step, (tv0, ti0))
    topv_ref[...] = tv
    topi_ref[...] = tix


def _threefry_bits(x1):
    ks = (_KS0, _KS1, _KS2)
    x0 = jnp.zeros_like(x1) + ks[0]
    x1 = x1 + ks[1]

    def rotl(v, r):
        return lax.shift_left(v, np.int32(r)) | lax.shift_right_logical(v, np.int32(32 - r))

    for g in range(5):
        for j in range(4):
            r = _ROTS[g * 4 + j]
            x0 = x0 + x1
            x1 = rotl(x1, r)
            x1 = x1 ^ x0
        x0 = x0 + ks[(g + 1) % 3]
        x1 = x1 + ks[(g + 2) % 3] + np.int32(g + 1)
    return x0 ^ x1


def _stage2_body(topv_ref, topi_ref, z_ref, out_ref):
    v = topv_ref[...]
    ti = topi_ref[...]
    kcol = lax.broadcasted_iota(jnp.int32, (_R, 128), 1)
    valid = kcol < _TOP_K
    m = v[:, 0:1]
    ex = jnp.where(valid, jnp.exp(v - m), np.float32(0.0))
    zv = z_ref[:, 0:1]
    p = ex / zv
    ri = lax.broadcasted_iota(jnp.int32, (128, 128), 0)
    ci = lax.broadcasted_iota(jnp.int32, (128, 128), 1)
    ltri = (ri < ci).astype(jnp.float32)
    cumprev = lax.dot_general(p, ltri, (((1,), (0,)), ((), ())),
                              precision=lax.Precision.HIGHEST,
                              preferred_element_type=jnp.float32)
    keep = ((cumprev <= _TOP_P) | (kcol == 0)) & valid
    s = jnp.sum(jnp.where(keep, ex, np.float32(0.0)), axis=-1, keepdims=True)
    logp = jnp.log(ex / s)
    row = lax.broadcasted_iota(jnp.int32, (_R, 128), 0)
    flat = row * np.int32(_V) + ti
    bits = _threefry_bits(flat)
    fbits = lax.shift_right_logical(bits, np.int32(9)) | np.int32(0x3F800000)
    frac = lax.bitcast_convert_type(fbits, jnp.float32) - np.float32(1.0)
    u = jnp.maximum(_TINY, frac * (np.float32(1.0) - _TINY) + _TINY)
    g = -jnp.log(-jnp.log(u))
    score = jnp.where(keep, logp + g, _NEG_INF)
    best = jnp.max(score, axis=-1, keepdims=True)
    tok = jnp.min(jnp.where(score == best, ti, jnp.int32(2 ** 30)),
                  axis=-1, keepdims=True)
    out_ref[...] = jnp.broadcast_to(tok, (_R, 128))


@jax.jit
def kernel(input_ids, logits):
    del input_ids
    topv, topi, z = pl.pallas_call(
        _stage1_body,
        grid=(_R // _RB,),
        in_specs=[pl.BlockSpec((_RB, _V), lambda i: (i, 0))],
        out_specs=[
            pl.BlockSpec((_RB, 128), lambda i: (i, 0)),
            pl.BlockSpec((_RB, 128), lambda i: (i, 0)),
            pl.BlockSpec((_RB, 128), lambda i: (i, 0)),
        ],
        out_shape=[
            jax.ShapeDtypeStruct((_R, 128), jnp.float32),
            jax.ShapeDtypeStruct((_R, 128), jnp.int32),
            jax.ShapeDtypeStruct((_R, 128), jnp.float32),
        ],
        scratch_shapes=[pltpu.VMEM((_RB, _V), jnp.float32)],
    )(logits)

    out = pl.pallas_call(
        _stage2_body,
        out_shape=jax.ShapeDtypeStruct((_R, 128), jnp.int32),
    )(topv, topi, z)
    return out[:, 0]


# speedup vs baseline: 74.0688x; 1.3955x over previous
import functools

import jax
import jax.numpy as jnp
from jax import lax
from jax.experimental import pallas as pl
from jax.experimental.pallas import tpu as pltpu
from jax.experimental.pallas import tpu_sc as plsc
import numpy as np

_TEMP = np.float32(0.7)
_TOP_P = np.float32(0.9)
_TOP_K = 50
_R = 128
_V = 100000
_RB = 8
_CAP = 256
_NVEC = _V // 16
_NEG_INF = np.float32(-np.inf)
_NW = 32
_ROWS_PER_W = _R // _NW

_KS0 = np.int32(0)
_KS1 = np.int32(42)
_KS2 = np.int32(0 ^ 42 ^ 0x1BD11BDA)
_ROTS = (13, 15, 26, 6, 17, 29, 16, 24, 13, 15, 26, 6, 17, 29, 16, 24, 13, 15, 26, 6)
_TINY = np.float32(np.finfo(np.float32).tiny)

_FULL = (_V // 128) * 128
_TAIL = _V - _FULL


def _pre_body(x_ref, z_ref, th_ref):
    def mstep(j, acc):
        start = pl.multiple_of(j * 128, 128)
        c = x_ref[:, pl.ds(start, 128)] / _TEMP
        return jnp.maximum(acc, c)

    acc0 = jnp.full((_RB, 128), _NEG_INF, jnp.float32)
    acc = lax.fori_loop(0, _FULL // 128, mstep, acc0)
    tail = x_ref[:, _FULL:_V] / _TEMP
    tailw = jnp.concatenate(
        [tail, jnp.full((_RB, 128 - _TAIL), _NEG_INF, jnp.float32)], axis=1)
    acc = jnp.maximum(acc, tailw)
    m = jnp.max(acc, axis=-1, keepdims=True)

    def zstep(j, zacc):
        start = pl.multiple_of(j * 128, 128)
        c = x_ref[:, pl.ds(start, 128)] / _TEMP
        return zacc + jnp.exp(c - m)

    zacc = lax.fori_loop(0, _FULL // 128, zstep, jnp.zeros((_RB, 128), jnp.float32))
    zacc = zacc + jnp.exp(tailw - m)
    z = jnp.sum(zacc, axis=-1, keepdims=True)
    z_ref[...] = jnp.broadcast_to(z, (_RB, 128))

    def tstep(k, carry):
        a, _ = carry
        mm = jnp.max(a, axis=-1, keepdims=True)
        return jnp.where(a == mm, _NEG_INF, a), mm

    _, th = lax.fori_loop(0, _TOP_K, tstep, (acc, m))
    th_ref[...] = jnp.broadcast_to(th, (_RB, 16))


def _sc_body(x_hbm, th_hbm, cv_hbm, ci_hbm, cnt_hbm,
             rowbuf, cv_v, ci_v, th_v, cnt_v):
    wid = lax.axis_index("s") * 2 + lax.axis_index("c")
    lane = lax.iota(jnp.int32, 16)
    for rr in range(_ROWS_PER_W):
        row = wid * _ROWS_PER_W + rr
        pltpu.sync_copy(x_hbm.at[pl.ds(row * _V, _V)], rowbuf)
        pltpu.sync_copy(th_hbm.at[pl.ds(row * 16, 16)], th_v)
        th = th_v[...]

        def body(j, carry):
            off, base = carry
            v = rowbuf[pl.ds(j * 16, 16)] / _TEMP
            msk = v >= th
            pc = plsc.all_reduce_population_count(msk)[0]

            @pl.when(pc > 0)
            def _append():
                kv = jnp.where(msk, v, _NEG_INF)
                iv = jnp.where(msk, base + lane, jnp.int32(0))
                sk, sv = plsc.sort_key_val(kv, iv, descending=True)
                cv_v[pl.ds(off, 16)] = sk
                ci_v[pl.ds(off, 16)] = sv

            off = jnp.minimum(off + pc, jnp.int32(_CAP))
            return off, base + jnp.int32(16)

        off, _ = lax.fori_loop(0, _NVEC, body, (jnp.int32(0), jnp.int32(0)))
        cnt_v[...] = jnp.full((16,), off, jnp.int32)
        pltpu.sync_copy(cv_v.at[pl.ds(0, _CAP)], cv_hbm.at[pl.ds(row * _CAP, _CAP)])
        pltpu.sync_copy(ci_v.at[pl.ds(0, _CAP)], ci_hbm.at[pl.ds(row * _CAP, _CAP)])
        pltpu.sync_copy(cnt_v, cnt_hbm.at[pl.ds(row * 16, 16)])


def _threefry_bits(x1):
    ks = (_KS0, _KS1, _KS2)
    x0 = jnp.zeros_like(x1) + ks[0]
    x1 = x1 + ks[1]

    def rotl(v, r):
        return lax.shift_left(v, np.int32(r)) | lax.shift_right_logical(v, np.int32(32 - r))

    for g in range(5):
        for j in range(4):
            r = _ROTS[g * 4 + j]
            x0 = x0 + x1
            x1 = rotl(x1, r)
            x1 = x1 ^ x0
        x0 = x0 + ks[(g + 1) % 3]
        x1 = x1 + ks[(g + 2) % 3] + np.int32(g + 1)
    return x0 ^ x1


def _fin_body(cv_ref, ci_ref, cnt_ref, z_ref, out_ref):
    cv = cv_ref[...]
    ci = ci_ref[...]
    n = cnt_ref[:, 0:1]
    lane_c = lax.broadcasted_iota(jnp.int32, (_R, _CAP), 1)
    cvm = jnp.where(lane_c < n, cv, _NEG_INF)
    kcol = lax.broadcasted_iota(jnp.int32, (_R, 128), 1)

    def estep(k, carry):
        cva, tv, tix = carry
        mm = jnp.max(cva, axis=-1, keepdims=True)
        tk = jnp.min(jnp.where(cva == mm, ci, jnp.int32(2 ** 30)),
                     axis=-1, keepdims=True)
        kmask = kcol == k
        tv = jnp.where(kmask, mm, tv)
        tix = jnp.where(kmask, tk, tix)
        cva = jnp.where(ci == tk, _NEG_INF, cva)
        return cva, tv, tix

    tv0 = jnp.full((_R, 128), _NEG_INF, jnp.float32)
    ti0 = jnp.zeros((_R, 128), jnp.int32)
    _, v, ti = lax.fori_loop(0, _TOP_K, estep, (cvm, tv0, ti0))

    valid = kcol < _TOP_K
    m = v[:, 0:1]
    ex = jnp.where(valid, jnp.exp(v - m), np.float32(0.0))
    zv = z_ref[:, 0:1]
    p = ex / zv
    ri = lax.broadcasted_iota(jnp.int32, (128, 128), 0)
    cicol = lax.broadcasted_iota(jnp.int32, (128, 128), 1)
    ltri = (ri < cicol).astype(jnp.float32)
    cumprev = lax.dot_general(p, ltri, (((1,), (0,)), ((), ())),
                              precision=lax.Precision.HIGHEST,
                              preferred_element_type=jnp.float32)
    keep = ((cumprev <= _TOP_P) | (kcol == 0)) & valid
    s = jnp.sum(jnp.where(keep, ex, np.float32(0.0)), axis=-1, keepdims=True)
    logp = jnp.log(ex / s)
    row = lax.broadcasted_iota(jnp.int32, (_R, 128), 0)
    flat = row * np.int32(_V) + ti
    bits = _threefry_bits(flat)
    fbits = lax.shift_right_logical(bits, np.int32(9)) | np.int32(0x3F800000)
    frac = lax.bitcast_convert_type(fbits, jnp.float32) - np.float32(1.0)
    u = jnp.maximum(_TINY, frac * (np.float32(1.0) - _TINY) + _TINY)
    g = -jnp.log(-jnp.log(u))
    score = jnp.where(keep, logp + g, _NEG_INF)
    best = jnp.max(score, axis=-1, keepdims=True)
    tok = jnp.min(jnp.where(score == best, ti, jnp.int32(2 ** 30)),
                  axis=-1, keepdims=True)
    out_ref[...] = jnp.broadcast_to(tok, (_R, 128))


_sc_compact = functools.partial(
    pl.kernel,
    out_type=[
        jax.ShapeDtypeStruct((_R * _CAP,), jnp.float32),
        jax.ShapeDtypeStruct((_R * _CAP,), jnp.int32),
        jax.ShapeDtypeStruct((_R * 16,), jnp.int32),
    ],
    mesh=plsc.VectorSubcoreMesh(core_axis_name="c", subcore_axis_name="s",
                                num_cores=2, num_subcores=16),
    scratch_types=[
        pltpu.VMEM((_V,), jnp.float32),
        pltpu.VMEM((_CAP + 16,), jnp.float32),
        pltpu.VMEM((_CAP + 16,), jnp.int32),
        pltpu.VMEM((16,), jnp.float32),
        pltpu.VMEM((16,), jnp.int32),
    ],
    compiler_params=pltpu.CompilerParams(needs_layout_passes=False),
)(_sc_body)


@jax.jit
def kernel(input_ids, logits):
    del input_ids
    z, th = pl.pallas_call(
        _pre_body,
        grid=(_R // _RB,),
        in_specs=[pl.BlockSpec((_RB, _V), lambda i: (i, 0))],
        out_specs=[
            pl.BlockSpec((_RB, 128), lambda i: (i, 0)),
            pl.BlockSpec((_RB, 16), lambda i: (i, 0)),
        ],
        out_shape=[
            jax.ShapeDtypeStruct((_R, 128), jnp.float32),
            jax.ShapeDtypeStruct((_R, 16), jnp.float32),
        ],
    )(logits)

    cvf, cif, cntf = _sc_compact(logits.reshape(-1), th.reshape(-1))

    out = pl.pallas_call(
        _fin_body,
        out_shape=jax.ShapeDtypeStruct((_R, 128), jnp.int32),
    )(cvf.reshape(_R, _CAP), cif.reshape(_R, _CAP),
      cntf.reshape(_R, 16), z)
    return out[:, 0]


# speedup vs baseline: 214.4928x; 2.8959x over previous
import functools

import jax
import jax.numpy as jnp
from jax import lax
from jax.experimental import pallas as pl
from jax.experimental.pallas import tpu as pltpu
from jax.experimental.pallas import tpu_sc as plsc
import numpy as np

_TEMP = np.float32(0.7)
_TOP_P = np.float32(0.9)
_TOP_K = 50
_R = 128
_V = 100000
_RB = 8
_CAP = 256
_NEG_INF = np.float32(-np.inf)
_NW = 32
_ROWS_PER_W = _R // _NW

_CH = 20000
_NCH = _V // _CH
_G = 5
_NGRP = _CH // (16 * _G)

_KS0 = np.int32(0)
_KS1 = np.int32(42)
_KS2 = np.int32(0 ^ 42 ^ 0x1BD11BDA)
_ROTS = (13, 15, 26, 6, 17, 29, 16, 24, 13, 15, 26, 6, 17, 29, 16, 24, 13, 15, 26, 6)
_TINY = np.float32(np.finfo(np.float32).tiny)

_UN = 11
_FULL = (_V // 128) * 128
_TAIL = _V - _FULL


def _pre_body(x_ref, z_ref, th_ref):
    def mstep(j, acc):
        for i in range(_UN):
            start = pl.multiple_of(j * (128 * _UN) + i * 128, 128)
            acc = jnp.maximum(acc, x_ref[:, pl.ds(start, 128)])
        return acc

    acc0 = jnp.full((_RB, 128), _NEG_INF, jnp.float32)
    acc = lax.fori_loop(0, _FULL // (128 * _UN), mstep, acc0)
    tail = x_ref[:, _FULL:_V]
    tailw = jnp.concatenate(
        [tail, jnp.full((_RB, 128 - _TAIL), _NEG_INF, jnp.float32)], axis=1)
    acc = jnp.maximum(acc, tailw)
    m = jnp.max(acc, axis=-1, keepdims=True) / _TEMP

    def zstep(j, zacc):
        for i in range(_UN):
            start = pl.multiple_of(j * (128 * _UN) + i * 128, 128)
            zacc = zacc + jnp.exp(x_ref[:, pl.ds(start, 128)] / _TEMP - m)
        return zacc

    zacc = lax.fori_loop(0, _FULL // (128 * _UN), zstep,
                         jnp.zeros((_RB, 128), jnp.float32))
    zacc = zacc + jnp.exp(tailw / _TEMP - m)
    z = jnp.sum(zacc, axis=-1, keepdims=True)
    z_ref[...] = jnp.broadcast_to(z, (_RB, 128))

    def tstep(k, carry):
        a, _ = carry
        mm = jnp.max(a, axis=-1, keepdims=True)
        return jnp.where(a == mm, _NEG_INF, a), mm

    _, th = lax.fori_loop(0, _TOP_K, tstep, (acc, acc0[:, 0:1]))
    th = th - jnp.abs(th) * np.float32(3e-6) - np.float32(1e-33)
    th_ref[...] = jnp.broadcast_to(th, (_RB, 16))


def _sc_body(x_hbm, th_hbm, cv_hbm, ci_hbm, cnt_hbm,
             buf0, buf1, cv_v, ci_v, th_v, cnt_v, cnt_smem, sem0, sem1):
    wid = lax.axis_index("s") * 2 + lax.axis_index("c")
    lane = lax.iota(jnp.int32, 16)
    row0 = wid * _ROWS_PER_W
    pltpu.sync_copy(th_hbm.at[pl.ds(row0 * 16, _ROWS_PER_W * 16)], th_v)

    bufs = (buf0, buf1)
    sems = (sem0, sem1)
    sched = [(rr, c) for rr in range(_ROWS_PER_W) for c in range(_NCH)]

    def issue(t):
        rr, c = sched[t]
        b = t % 2
        return pltpu.async_copy(
            x_hbm.at[pl.ds((row0 + rr) * _V + c * _CH, _CH)], bufs[b], sems[b])

    handle = issue(0)
    for t, (rr, c) in enumerate(sched):
        nxt = issue(t + 1) if t + 1 < len(sched) else None
        handle.wait()
        buf = bufs[t % 2]
        th = th_v[pl.ds(rr * 16, 16)]
        if c == 0:
            cnt_smem[0] = jnp.int32(0)

        def group(j, _, buf=buf, th=th, base_c=c * _CH):
            vs = [buf[pl.ds(j * (16 * _G) + i * 16, 16)] for i in range(_G)]
            ms = [v >= th for v in vs]
            orm = ms[0]
            for i in range(1, _G):
                orm = orm | ms[i]
            pcg = plsc.all_reduce_population_count(orm)[0]

            @pl.when(pcg > 0)
            def _slow():
                for i in range(_G):
                    pci = plsc.all_reduce_population_count(ms[i])[0]

                    def _append(i=i, pci=pci):
                        off = cnt_smem[0]
                        soff = jnp.minimum(off, jnp.int32(_CAP))
                        kv = jnp.where(ms[i], vs[i] / _TEMP, _NEG_INF)
                        iv = jnp.where(
                            ms[i],
                            base_c + j * (16 * _G) + np.int32(i * 16) + lane,
                            jnp.int32(0))
                        sk, sv = plsc.sort_key_val(kv, iv, descending=True)
                        cv_v[pl.ds(soff, 16)] = sk
                        ci_v[pl.ds(soff, 16)] = sv
                        cnt_smem[0] = off + pci

                    pl.when(pci > 0)(_append)

            return 0

        lax.fori_loop(0, _NGRP, group, 0)

        if c == _NCH - 1:
            row = row0 + rr
            off = jnp.minimum(cnt_smem[0], jnp.int32(_CAP))
            cnt_v[...] = jnp.full((16,), off, jnp.int32)
            pltpu.sync_copy(cv_v.at[pl.ds(0, _CAP)],
                            cv_hbm.at[pl.ds(row * _CAP, _CAP)])
            pltpu.sync_copy(ci_v.at[pl.ds(0, _CAP)],
                            ci_hbm.at[pl.ds(row * _CAP, _CAP)])
            pltpu.sync_copy(cnt_v, cnt_hbm.at[pl.ds(row * 16, 16)])
        handle = nxt


def _threefry_bits(x1):
    ks = (_KS0, _KS1, _KS2)
    x0 = jnp.zeros_like(x1) + ks[0]
    x1 = x1 + ks[1]

    def rotl(v, r):
        return lax.shift_left(v, np.int32(r)) | lax.shift_right_logical(v, np.int32(32 - r))

    for g in range(5):
        for j in range(4):
            r = _ROTS[g * 4 + j]
            x0 = x0 + x1
            x1 = rotl(x1, r)
            x1 = x1 ^ x0
        x0 = x0 + ks[(g + 1) % 3]
        x1 = x1 + ks[(g + 2) % 3] + np.int32(g + 1)
    return x0 ^ x1


def _fin_body(cv_ref, ci_ref, cnt_ref, z_ref, out_ref):
    cv = cv_ref[...]
    ci = ci_ref[...]
    n = cnt_ref[:, 0:1]
    lane_c = lax.broadcasted_iota(jnp.int32, (_R, _CAP), 1)
    cvm = jnp.where(lane_c < n, cv, _NEG_INF)
    kcol = lax.broadcasted_iota(jnp.int32, (_R, 128), 1)

    def estep(k, carry):
        cva, tv, tix = carry
        mm = jnp.max(cva, axis=-1, keepdims=True)
        tk = jnp.min(jnp.where(cva == mm, ci, jnp.int32(2 ** 30)),
                     axis=-1, keepdims=True)
        kmask = kcol == k
        tv = jnp.where(kmask, mm, tv)
        tix = jnp.where(kmask, tk, tix)
        cva = jnp.where(ci == tk, _NEG_INF, cva)
        return cva, tv, tix

    tv0 = jnp.full((_R, 128), _NEG_INF, jnp.float32)
    ti0 = jnp.zeros((_R, 128), jnp.int32)
    _, v, ti = lax.fori_loop(0, _TOP_K, estep, (cvm, tv0, ti0))

    valid = kcol < _TOP_K
    m = v[:, 0:1]
    ex = jnp.where(valid, jnp.exp(v - m), np.float32(0.0))
    zv = z_ref[:, 0:1]
    p = ex / zv
    ri = lax.broadcasted_iota(jnp.int32, (128, 128), 0)
    cicol = lax.broadcasted_iota(jnp.int32, (128, 128), 1)
    ltri = (ri < cicol).astype(jnp.float32)
    cumprev = lax.dot_general(p, ltri, (((1,), (0,)), ((), ())),
                              precision=lax.Precision.HIGHEST,
                              preferred_element_type=jnp.float32)
    keep = ((cumprev <= _TOP_P) | (kcol == 0)) & valid
    s = jnp.sum(jnp.where(keep, ex, np.float32(0.0)), axis=-1, keepdims=True)
    logp = jnp.log(ex / s)
    row = lax.broadcasted_iota(jnp.int32, (_R, 128), 0)
    flat = row * np.int32(_V) + ti
    bits = _threefry_bits(flat)
    fbits = lax.shift_right_logical(bits, np.int32(9)) | np.int32(0x3F800000)
    frac = lax.bitcast_convert_type(fbits, jnp.float32) - np.float32(1.0)
    u = jnp.maximum(_TINY, frac * (np.float32(1.0) - _TINY) + _TINY)
    g = -jnp.log(-jnp.log(u))
    score = jnp.where(keep, logp + g, _NEG_INF)
    best = jnp.max(score, axis=-1, keepdims=True)
    tok = jnp.min(jnp.where(score == best, ti, jnp.int32(2 ** 30)),
                  axis=-1, keepdims=True)
    out_ref[...] = jnp.broadcast_to(tok, (_R, 128))


_sc_compact = functools.partial(
    pl.kernel,
    out_type=[
        jax.ShapeDtypeStruct((_R * _CAP,), jnp.float32),
        jax.ShapeDtypeStruct((_R * _CAP,), jnp.int32),
        jax.ShapeDtypeStruct((_R * 16,), jnp.int32),
    ],
    mesh=plsc.VectorSubcoreMesh(core_axis_name="c", subcore_axis_name="s",
                                num_cores=2, num_subcores=16),
    scratch_types=[
        pltpu.VMEM((_CH,), jnp.float32),
        pltpu.VMEM((_CH,), jnp.float32),
        pltpu.VMEM((_CAP + 16,), jnp.float32),
        pltpu.VMEM((_CAP + 16,), jnp.int32),
        pltpu.VMEM((_ROWS_PER_W * 16,), jnp.float32),
        pltpu.VMEM((16,), jnp.int32),
        pltpu.SMEM((1,), jnp.int32),
        pltpu.SemaphoreType.DMA,
        pltpu.SemaphoreType.DMA,
    ],
    compiler_params=pltpu.CompilerParams(needs_layout_passes=False),
)(_sc_body)


@jax.jit
def kernel(input_ids, logits):
    del input_ids
    z, th = pl.pallas_call(
        _pre_body,
        grid=(_R // _RB,),
        in_specs=[pl.BlockSpec((_RB, _V), lambda i: (i, 0))],
        out_specs=[
            pl.BlockSpec((_RB, 128), lambda i: (i, 0)),
            pl.BlockSpec((_RB, 16), lambda i: (i, 0)),
        ],
        out_shape=[
            jax.ShapeDtypeStruct((_R, 128), jnp.float32),
            jax.ShapeDtypeStruct((_R, 16), jnp.float32),
        ],
    )(logits)

    cvf, cif, cntf = _sc_compact(logits.reshape(-1), th.reshape(-1))

    out = pl.pallas_call(
        _fin_body,
        out_shape=jax.ShapeDtypeStruct((_R, 128), jnp.int32),
    )(cvf.reshape(_R, _CAP), cif.reshape(_R, _CAP),
      cntf.reshape(_R, 16), z)
    return out[:, 0]


# speedup vs baseline: 214.6833x; 1.0009x over previous
import functools

import jax
import jax.numpy as jnp
from jax import lax
from jax.experimental import pallas as pl
from jax.experimental.pallas import tpu as pltpu
from jax.experimental.pallas import tpu_sc as plsc
import numpy as np

_TEMP = np.float32(0.7)
_TOP_P = np.float32(0.9)
_TOP_K = 50
_R = 128
_V = 100000
_RB = 8
_CAP = 256
_NEG_INF = np.float32(-np.inf)
_NW = 32
_ROWS_PER_W = _R // _NW

_CH = 20000
_NCH = _V // _CH
_G = 5
_NGRP = _CH // (16 * _G)

_KS0 = np.int32(0)
_KS1 = np.int32(42)
_KS2 = np.int32(0 ^ 42 ^ 0x1BD11BDA)
_ROTS = (13, 15, 26, 6, 17, 29, 16, 24, 13, 15, 26, 6, 17, 29, 16, 24, 13, 15, 26, 6)
_TINY = np.float32(np.finfo(np.float32).tiny)

_UN = 11
_FULL = (_V // 128) * 128
_TAIL = _V - _FULL


def _pre_body(x_ref, z_ref, th_ref):
    def mstep(j, accs):
        accs = list(accs)
        for i in range(_UN):
            start = pl.multiple_of(j * (128 * _UN) + i * 128, 128)
            accs[i % 4] = jnp.maximum(accs[i % 4], x_ref[:, pl.ds(start, 128)])
        return tuple(accs)

    acc0 = jnp.full((_RB, 128), _NEG_INF, jnp.float32)
    accs = lax.fori_loop(0, _FULL // (128 * _UN), mstep, (acc0,) * 4)
    tail = x_ref[:, _FULL:_V]
    tailw = jnp.concatenate(
        [tail, jnp.full((_RB, 128 - _TAIL), _NEG_INF, jnp.float32)], axis=1)
    acc = jnp.maximum(jnp.maximum(accs[0], accs[1]),
                      jnp.maximum(jnp.maximum(accs[2], accs[3]), tailw))
    m = jnp.max(acc, axis=-1, keepdims=True) / _TEMP

    inv = np.float32(1.0) / _TEMP

    def zstep(j, zaccs):
        zaccs = list(zaccs)
        for i in range(_UN):
            start = pl.multiple_of(j * (128 * _UN) + i * 128, 128)
            zaccs[i % 4] = zaccs[i % 4] + jnp.exp(
                x_ref[:, pl.ds(start, 128)] * inv - m)
        return tuple(zaccs)

    zacc0 = jnp.zeros((_RB, 128), jnp.float32)
    zaccs = lax.fori_loop(0, _FULL // (128 * _UN), zstep, (zacc0,) * 4)
    zacc = (zaccs[0] + zaccs[1]) + (zaccs[2] + zaccs[3]) + jnp.exp(tailw * inv - m)
    z = jnp.sum(zacc, axis=-1, keepdims=True)
    z_ref[...] = jnp.broadcast_to(z, (_RB, 128))

    def tstep(k, carry):
        a, _ = carry
        mm = jnp.max(a, axis=-1, keepdims=True)
        return jnp.where(a == mm, _NEG_INF, a), mm

    _, th = lax.fori_loop(0, _TOP_K, tstep, (acc, acc0[:, 0:1]))
    th = th - jnp.abs(th) * np.float32(3e-6) - np.float32(1e-33)
    th_ref[...] = jnp.broadcast_to(th, (_RB, 16))


def _sc_body(x_hbm, th_hbm, cv_hbm, ci_hbm, cnt_hbm,
             buf0, buf1, cv_v, ci_v, th_v, cnt_v, cnt_smem, sem0, sem1):
    wid = lax.axis_index("s") * 2 + lax.axis_index("c")
    lane = lax.iota(jnp.int32, 16)
    row0 = wid * _ROWS_PER_W
    pltpu.sync_copy(th_hbm.at[pl.ds(row0 * 16, _ROWS_PER_W * 16)], th_v)

    bufs = (buf0, buf1)
    sems = (sem0, sem1)
    sched = [(rr, c) for rr in range(_ROWS_PER_W) for c in range(_NCH)]

    def issue(t):
        rr, c = sched[t]
        b = t % 2
        return pltpu.async_copy(
            x_hbm.at[pl.ds((row0 + rr) * _V + c * _CH, _CH)], bufs[b], sems[b])

    handle = issue(0)
    for t, (rr, c) in enumerate(sched):
        nxt = issue(t + 1) if t + 1 < len(sched) else None
        handle.wait()
        buf = bufs[t % 2]
        th = th_v[pl.ds(rr * 16, 16)]
        if c == 0:
            cnt_smem[0] = jnp.int32(0)

        def group(j, _, buf=buf, th=th, base_c=c * _CH):
            vs = [buf[pl.ds(j * (16 * _G) + i * 16, 16)] for i in range(_G)]
            ms = [v >= th for v in vs]
            orm = ms[0]
            for i in range(1, _G):
                orm = orm | ms[i]
            pcg = plsc.all_reduce_population_count(orm)[0]

            @pl.when(pcg > 0)
            def _slow():
                for i in range(_G):
                    pci = plsc.all_reduce_population_count(ms[i])[0]

                    def _append(i=i, pci=pci):
                        off = cnt_smem[0]
                        soff = jnp.minimum(off, jnp.int32(_CAP))
                        kv = jnp.where(ms[i], vs[i] / _TEMP, _NEG_INF)
                        iv = jnp.where(
                            ms[i],
                            base_c + j * (16 * _G) + np.int32(i * 16) + lane,
                            jnp.int32(0))
                        sk, sv = plsc.sort_key_val(kv, iv, descending=True)
                        cv_v[pl.ds(soff, 16)] = sk
                        ci_v[pl.ds(soff, 16)] = sv
                        cnt_smem[0] = off + pci

                    pl.when(pci > 0)(_append)

            return 0

        lax.fori_loop(0, _NGRP, group, 0)

        if c == _NCH - 1:
            row = row0 + rr
            off = jnp.minimum(cnt_smem[0], jnp.int32(_CAP))
            cnt_v[...] = jnp.full((16,), off, jnp.int32)
            pltpu.sync_copy(cv_v.at[pl.ds(0, _CAP)],
                            cv_hbm.at[pl.ds(row * _CAP, _CAP)])
            pltpu.sync_copy(ci_v.at[pl.ds(0, _CAP)],
                            ci_hbm.at[pl.ds(row * _CAP, _CAP)])
            pltpu.sync_copy(cnt_v, cnt_hbm.at[pl.ds(row * 16, 16)])
        handle = nxt


def _threefry_bits(x1):
    ks = (_KS0, _KS1, _KS2)
    x0 = jnp.zeros_like(x1) + ks[0]
    x1 = x1 + ks[1]

    def rotl(v, r):
        return lax.shift_left(v, np.int32(r)) | lax.shift_right_logical(v, np.int32(32 - r))

    for g in range(5):
        for j in range(4):
            r = _ROTS[g * 4 + j]
            x0 = x0 + x1
            x1 = rotl(x1, r)
            x1 = x1 ^ x0
        x0 = x0 + ks[(g + 1) % 3]
        x1 = x1 + ks[(g + 2) % 3] + np.int32(g + 1)
    return x0 ^ x1


def _fin_body(cv_ref, ci_ref, cnt_ref, z_ref, out_ref):
    cv = cv_ref[...]
    ci = ci_ref[...]
    n = cnt_ref[:, 0:1]
    lane_c = lax.broadcasted_iota(jnp.int32, (_R, _CAP), 1)
    cvm = jnp.where(lane_c < n, cv, _NEG_INF)
    kcol = lax.broadcasted_iota(jnp.int32, (_R, 128), 1)

    def estep(k, carry):
        cva, tv, tix = carry
        mm = jnp.max(cva, axis=-1, keepdims=True)
        tk = jnp.min(jnp.where(cva == mm, ci, jnp.int32(2 ** 30)),
                     axis=-1, keepdims=True)
        kmask = kcol == k
        tv = jnp.where(kmask, mm, tv)
        tix = jnp.where(kmask, tk, tix)
        cva = jnp.where(ci == tk, _NEG_INF, cva)
        return cva, tv, tix

    tv0 = jnp.full((_R, 128), _NEG_INF, jnp.float32)
    ti0 = jnp.zeros((_R, 128), jnp.int32)
    _, v, ti = lax.fori_loop(0, _TOP_K, estep, (cvm, tv0, ti0))

    valid = kcol < _TOP_K
    m = v[:, 0:1]
    ex = jnp.where(valid, jnp.exp(v - m), np.float32(0.0))
    zv = z_ref[:, 0:1]
    p = ex / zv
    ri = lax.broadcasted_iota(jnp.int32, (128, 128), 0)
    cicol = lax.broadcasted_iota(jnp.int32, (128, 128), 1)
    ltri = (ri < cicol).astype(jnp.float32)
    cumprev = lax.dot_general(p, ltri, (((1,), (0,)), ((), ())),
                              precision=lax.Precision.HIGHEST,
                              preferred_element_type=jnp.float32)
    keep = ((cumprev <= _TOP_P) | (kcol == 0)) & valid
    s = jnp.sum(jnp.where(keep, ex, np.float32(0.0)), axis=-1, keepdims=True)
    logp = jnp.log(ex / s)
    row = lax.broadcasted_iota(jnp.int32, (_R, 128), 0)
    flat = row * np.int32(_V) + ti
    bits = _threefry_bits(flat)
    fbits = lax.shift_right_logical(bits, np.int32(9)) | np.int32(0x3F800000)
    frac = lax.bitcast_convert_type(fbits, jnp.float32) - np.float32(1.0)
    u = jnp.maximum(_TINY, frac * (np.float32(1.0) - _TINY) + _TINY)
    g = -jnp.log(-jnp.log(u))
    score = jnp.where(keep, logp + g, _NEG_INF)
    best = jnp.max(score, axis=-1, keepdims=True)
    tok = jnp.min(jnp.where(score == best, ti, jnp.int32(2 ** 30)),
                  axis=-1, keepdims=True)
    out_ref[...] = jnp.broadcast_to(tok, (_R, 128))


_sc_compact = functools.partial(
    pl.kernel,
    out_type=[
        jax.ShapeDtypeStruct((_R * _CAP,), jnp.float32),
        jax.ShapeDtypeStruct((_R * _CAP,), jnp.int32),
        jax.ShapeDtypeStruct((_R * 16,), jnp.int32),
    ],
    mesh=plsc.VectorSubcoreMesh(core_axis_name="c", subcore_axis_name="s",
                                num_cores=2, num_subcores=16),
    scratch_types=[
        pltpu.VMEM((_CH,), jnp.float32),
        pltpu.VMEM((_CH,), jnp.float32),
        pltpu.VMEM((_CAP + 16,), jnp.float32),
        pltpu.VMEM((_CAP + 16,), jnp.int32),
        pltpu.VMEM((_ROWS_PER_W * 16,), jnp.float32),
        pltpu.VMEM((16,), jnp.int32),
        pltpu.SMEM((1,), jnp.int32),
        pltpu.SemaphoreType.DMA,
        pltpu.SemaphoreType.DMA,
    ],
    compiler_params=pltpu.CompilerParams(needs_layout_passes=False),
)(_sc_body)


@jax.jit
def kernel(input_ids, logits):
    del input_ids
    z, th = pl.pallas_call(
        _pre_body,
        grid=(_R // _RB,),
        in_specs=[pl.BlockSpec((_RB, _V), lambda i: (i, 0))],
        out_specs=[
            pl.BlockSpec((_RB, 128), lambda i: (i, 0)),
            pl.BlockSpec((_RB, 16), lambda i: (i, 0)),
        ],
        out_shape=[
            jax.ShapeDtypeStruct((_R, 128), jnp.float32),
            jax.ShapeDtypeStruct((_R, 16), jnp.float32),
        ],
    )(logits)

    cvf, cif, cntf = _sc_compact(logits.reshape(-1), th.reshape(-1))

    out = pl.pallas_call(
        _fin_body,
        out_shape=jax.ShapeDtypeStruct((_R, 128), jnp.int32),
    )(cvf.reshape(_R, _CAP), cif.reshape(_R, _CAP),
      cntf.reshape(_R, 16), z)
    return out[:, 0]


# speedup vs baseline: 264.8914x; 1.2339x over previous
import functools

import jax
import jax.numpy as jnp
from jax import lax
from jax.experimental import pallas as pl
from jax.experimental.pallas import tpu as pltpu
from jax.experimental.pallas import tpu_sc as plsc
import numpy as np

_TEMP = np.float32(0.7)
_TOP_P = np.float32(0.9)
_TOP_K = 50
_R = 128
_V = 100000
_RB = 64
_CAP = 256
_NEG_INF = np.float32(-np.inf)
_NW = 32
_ROWS_PER_W = _R // _NW

_CH = 20000
_NCH = _V // _CH
_G = 5
_NGRP = _CH // (16 * _G)

_KS0 = np.int32(0)
_KS1 = np.int32(42)
_KS2 = np.int32(0 ^ 42 ^ 0x1BD11BDA)
_ROTS = (13, 15, 26, 6, 17, 29, 16, 24, 13, 15, 26, 6, 17, 29, 16, 24, 13, 15, 26, 6)
_TINY = np.float32(np.finfo(np.float32).tiny)

_UN = 11
_FULL = (_V // 128) * 128
_TAIL = _V - _FULL


def _pre_body(x_ref, z_ref, th_ref):
    def mstep(j, accs):
        accs = list(accs)
        for i in range(_UN):
            start = pl.multiple_of(j * (128 * _UN) + i * 128, 128)
            accs[i % 4] = jnp.maximum(accs[i % 4], x_ref[:, pl.ds(start, 128)])
        return tuple(accs)

    acc0 = jnp.full((_RB, 128), _NEG_INF, jnp.float32)
    accs = lax.fori_loop(0, _FULL // (128 * _UN), mstep, (acc0,) * 4)
    tail = x_ref[:, _FULL:_V]
    tailw = jnp.concatenate(
        [tail, jnp.full((_RB, 128 - _TAIL), _NEG_INF, jnp.float32)], axis=1)
    acc = jnp.maximum(jnp.maximum(accs[0], accs[1]),
                      jnp.maximum(jnp.maximum(accs[2], accs[3]), tailw))
    m = jnp.max(acc, axis=-1, keepdims=True) / _TEMP

    inv = np.float32(1.0) / _TEMP

    def zstep(j, zaccs):
        zaccs = list(zaccs)
        for i in range(_UN):
            start = pl.multiple_of(j * (128 * _UN) + i * 128, 128)
            zaccs[i % 4] = zaccs[i % 4] + jnp.exp(
                x_ref[:, pl.ds(start, 128)] * inv - m)
        return tuple(zaccs)

    zacc0 = jnp.zeros((_RB, 128), jnp.float32)
    zaccs = lax.fori_loop(0, _FULL // (128 * _UN), zstep, (zacc0,) * 4)
    zacc = (zaccs[0] + zaccs[1]) + (zaccs[2] + zaccs[3]) + jnp.exp(tailw * inv - m)
    z = jnp.sum(zacc, axis=-1, keepdims=True)
    z_ref[...] = jnp.broadcast_to(z, (_RB, 128))

    def tstep(k, carry):
        a, _ = carry
        mm = jnp.max(a, axis=-1, keepdims=True)
        return jnp.where(a == mm, _NEG_INF, a), mm

    _, th = lax.fori_loop(0, _TOP_K, tstep, (acc, acc0[:, 0:1]))
    th = th - jnp.abs(th) * np.float32(3e-6) - np.float32(1e-33)
    th_ref[...] = jnp.broadcast_to(th, (_RB, 16))


def _sc_body(x_hbm, th_hbm, cv_hbm, ci_hbm, cnt_hbm,
             buf0, buf1, cv_v, ci_v, th_v, cnt_v, cnt_smem, sem0, sem1):
    wid = lax.axis_index("s") * 2 + lax.axis_index("c")
    lane = lax.iota(jnp.int32, 16)
    row0 = wid * _ROWS_PER_W
    pltpu.sync_copy(th_hbm.at[pl.ds(row0 * 16, _ROWS_PER_W * 16)], th_v)

    bufs = (buf0, buf1)
    sems = (sem0, sem1)
    sched = [(rr, c) for rr in range(_ROWS_PER_W) for c in range(_NCH)]

    def issue(t):
        rr, c = sched[t]
        b = t % 2
        return pltpu.async_copy(
            x_hbm.at[pl.ds((row0 + rr) * _V + c * _CH, _CH)], bufs[b], sems[b])

    handle = issue(0)
    for t, (rr, c) in enumerate(sched):
        nxt = issue(t + 1) if t + 1 < len(sched) else None
        handle.wait()
        buf = bufs[t % 2]
        th = th_v[pl.ds(rr * 16, 16)]
        if c == 0:
            cnt_smem[0] = jnp.int32(0)

        def group(j, _, buf=buf, th=th, base_c=c * _CH):
            vs = [buf[pl.ds(j * (16 * _G) + i * 16, 16)] for i in range(_G)]
            ms = [v >= th for v in vs]
            orm = ms[0]
            for i in range(1, _G):
                orm = orm | ms[i]
            pcg = plsc.all_reduce_population_count(orm)[0]

            @pl.when(pcg > 0)
            def _slow():
                for i in range(_G):
                    pci = plsc.all_reduce_population_count(ms[i])[0]

                    def _append(i=i, pci=pci):
                        off = cnt_smem[0]
                        soff = jnp.minimum(off, jnp.int32(_CAP))
                        kv = jnp.where(ms[i], vs[i] / _TEMP, _NEG_INF)
                        iv = jnp.where(
                            ms[i],
                            base_c + j * (16 * _G) + np.int32(i * 16) + lane,
                            jnp.int32(0))
                        sk, sv = plsc.sort_key_val(kv, iv, descending=True)
                        cv_v[pl.ds(soff, 16)] = sk
                        ci_v[pl.ds(soff, 16)] = sv
                        cnt_smem[0] = off + pci

                    pl.when(pci > 0)(_append)

            return 0

        lax.fori_loop(0, _NGRP, group, 0)

        if c == _NCH - 1:
            row = row0 + rr
            off = jnp.minimum(cnt_smem[0], jnp.int32(_CAP))
            cnt_v[...] = jnp.full((16,), off, jnp.int32)
            pltpu.sync_copy(cv_v.at[pl.ds(0, _CAP)],
                            cv_hbm.at[pl.ds(row * _CAP, _CAP)])
            pltpu.sync_copy(ci_v.at[pl.ds(0, _CAP)],
                            ci_hbm.at[pl.ds(row * _CAP, _CAP)])
            pltpu.sync_copy(cnt_v, cnt_hbm.at[pl.ds(row * 16, 16)])
        handle = nxt


def _threefry_bits(x1):
    ks = (_KS0, _KS1, _KS2)
    x0 = jnp.zeros_like(x1) + ks[0]
    x1 = x1 + ks[1]

    def rotl(v, r):
        return lax.shift_left(v, np.int32(r)) | lax.shift_right_logical(v, np.int32(32 - r))

    for g in range(5):
        for j in range(4):
            r = _ROTS[g * 4 + j]
            x0 = x0 + x1
            x1 = rotl(x1, r)
            x1 = x1 ^ x0
        x0 = x0 + ks[(g + 1) % 3]
        x1 = x1 + ks[(g + 2) % 3] + np.int32(g + 1)
    return x0 ^ x1


def _fin_body(cv_ref, ci_ref, cnt_ref, z_ref, out_ref):
    cv = cv_ref[...]
    ci = ci_ref[...]
    n = cnt_ref[:, 0:1]
    lane_c = lax.broadcasted_iota(jnp.int32, (_R, _CAP), 1)
    cvm = jnp.where(lane_c < n, cv, _NEG_INF)
    kcol = lax.broadcasted_iota(jnp.int32, (_R, 128), 1)

    def estep(k, carry):
        cva, tv, tix = carry
        mm = jnp.max(cva, axis=-1, keepdims=True)
        tk = jnp.min(jnp.where(cva == mm, ci, jnp.int32(2 ** 30)),
                     axis=-1, keepdims=True)
        kmask = kcol == k
        tv = jnp.where(kmask, mm, tv)
        tix = jnp.where(kmask, tk, tix)
        cva = jnp.where(ci == tk, _NEG_INF, cva)
        return cva, tv, tix

    tv0 = jnp.full((_R, 128), _NEG_INF, jnp.float32)
    ti0 = jnp.zeros((_R, 128), jnp.int32)
    _, v, ti = lax.fori_loop(0, _TOP_K, estep, (cvm, tv0, ti0))

    valid = kcol < _TOP_K
    m = v[:, 0:1]
    ex = jnp.where(valid, jnp.exp(v - m), np.float32(0.0))
    zv = z_ref[:, 0:1]
    p = ex / zv
    ri = lax.broadcasted_iota(jnp.int32, (128, 128), 0)
    cicol = lax.broadcasted_iota(jnp.int32, (128, 128), 1)
    ltri = (ri < cicol).astype(jnp.float32)
    cumprev = lax.dot_general(p, ltri, (((1,), (0,)), ((), ())),
                              precision=lax.Precision.HIGHEST,
                              preferred_element_type=jnp.float32)
    keep = ((cumprev <= _TOP_P) | (kcol == 0)) & valid
    s = jnp.sum(jnp.where(keep, ex, np.float32(0.0)), axis=-1, keepdims=True)
    logp = jnp.log(ex / s)
    row = lax.broadcasted_iota(jnp.int32, (_R, 128), 0)
    flat = row * np.int32(_V) + ti
    bits = _threefry_bits(flat)
    fbits = lax.shift_right_logical(bits, np.int32(9)) | np.int32(0x3F800000)
    frac = lax.bitcast_convert_type(fbits, jnp.float32) - np.float32(1.0)
    u = jnp.maximum(_TINY, frac * (np.float32(1.0) - _TINY) + _TINY)
    g = -jnp.log(-jnp.log(u))
    score = jnp.where(keep, logp + g, _NEG_INF)
    best = jnp.max(score, axis=-1, keepdims=True)
    tok = jnp.min(jnp.where(score == best, ti, jnp.int32(2 ** 30)),
                  axis=-1, keepdims=True)
    out_ref[...] = jnp.broadcast_to(tok, (_R, 128))


_sc_compact = functools.partial(
    pl.kernel,
    out_type=[
        jax.ShapeDtypeStruct((_R * _CAP,), jnp.float32),
        jax.ShapeDtypeStruct((_R * _CAP,), jnp.int32),
        jax.ShapeDtypeStruct((_R * 16,), jnp.int32),
    ],
    mesh=plsc.VectorSubcoreMesh(core_axis_name="c", subcore_axis_name="s",
                                num_cores=2, num_subcores=16),
    scratch_types=[
        pltpu.VMEM((_CH,), jnp.float32),
        pltpu.VMEM((_CH,), jnp.float32),
        pltpu.VMEM((_CAP + 16,), jnp.float32),
        pltpu.VMEM((_CAP + 16,), jnp.int32),
        pltpu.VMEM((_ROWS_PER_W * 16,), jnp.float32),
        pltpu.VMEM((16,), jnp.int32),
        pltpu.SMEM((1,), jnp.int32),
        pltpu.SemaphoreType.DMA,
        pltpu.SemaphoreType.DMA,
    ],
    compiler_params=pltpu.CompilerParams(needs_layout_passes=False),
)(_sc_body)


@jax.jit
def kernel(input_ids, logits):
    del input_ids
    z, th = pl.pallas_call(
        _pre_body,
        grid=(_R // _RB,),
        in_specs=[pl.BlockSpec((_RB, _V), lambda i: (i, 0))],
        out_specs=[
            pl.BlockSpec((_RB, 128), lambda i: (i, 0)),
            pl.BlockSpec((_RB, 16), lambda i: (i, 0)),
        ],
        out_shape=[
            jax.ShapeDtypeStruct((_R, 128), jnp.float32),
            jax.ShapeDtypeStruct((_R, 16), jnp.float32),
        ],
    )(logits)

    cvf, cif, cntf = _sc_compact(logits.reshape(-1), th.reshape(-1))

    out = pl.pallas_call(
        _fin_body,
        out_shape=jax.ShapeDtypeStruct((_R, 128), jnp.int32),
    )(cvf.reshape(_R, _CAP), cif.reshape(_R, _CAP),
      cntf.reshape(_R, 16), z)
    return out[:, 0]


# speedup vs baseline: 321.1809x; 1.2125x over previous
import functools

import jax
import jax.numpy as jnp
from jax import lax
from jax.experimental import pallas as pl
from jax.experimental.pallas import tpu as pltpu
from jax.experimental.pallas import tpu_sc as plsc
import numpy as np

_TEMP = np.float32(0.7)
_TOP_P = np.float32(0.9)
_TOP_K = 50
_R = 128
_V = 100000
_RB = 64
_CAP = 256
_NEG_INF = np.float32(-np.inf)
_NW = 32
_ROWS_PER_W = _R // _NW

_CH = 20000
_NCH = _V // _CH
_G = 10
_NGRP = _CH // (16 * _G)

_KS0 = np.int32(0)
_KS1 = np.int32(42)
_KS2 = np.int32(0 ^ 42 ^ 0x1BD11BDA)
_ROTS = (13, 15, 26, 6, 17, 29, 16, 24, 13, 15, 26, 6, 17, 29, 16, 24, 13, 15, 26, 6)
_TINY = np.float32(np.finfo(np.float32).tiny)

_UN = 11
_FULL = (_V // 128) * 128
_TAIL = _V - _FULL


def _pre_body(x_ref, z_ref, th_ref):
    def mstep(j, accs):
        accs = list(accs)
        for i in range(_UN):
            start = pl.multiple_of(j * (128 * _UN) + i * 128, 128)
            accs[i % 4] = jnp.maximum(accs[i % 4], x_ref[:, pl.ds(start, 128)])
        return tuple(accs)

    acc0 = jnp.full((_RB, 128), _NEG_INF, jnp.float32)
    accs = lax.fori_loop(0, _FULL // (128 * _UN), mstep, (acc0,) * 4)
    tail = x_ref[:, _FULL:_V]
    tailw = jnp.concatenate(
        [tail, jnp.full((_RB, 128 - _TAIL), _NEG_INF, jnp.float32)], axis=1)
    acc = jnp.maximum(jnp.maximum(accs[0], accs[1]),
                      jnp.maximum(jnp.maximum(accs[2], accs[3]), tailw))
    m = jnp.max(acc, axis=-1, keepdims=True) / _TEMP

    inv = np.float32(1.0) / _TEMP

    def zstep(j, zaccs):
        zaccs = list(zaccs)
        for i in range(_UN):
            start = pl.multiple_of(j * (128 * _UN) + i * 128, 128)
            zaccs[i % 4] = zaccs[i % 4] + jnp.exp(
                x_ref[:, pl.ds(start, 128)] * inv - m)
        return tuple(zaccs)

    zacc0 = jnp.zeros((_RB, 128), jnp.float32)
    zaccs = lax.fori_loop(0, _FULL // (128 * _UN), zstep, (zacc0,) * 4)
    zacc = (zaccs[0] + zaccs[1]) + (zaccs[2] + zaccs[3]) + jnp.exp(tailw * inv - m)
    z = jnp.sum(zacc, axis=-1, keepdims=True)
    z_ref[...] = jnp.broadcast_to(z, (_RB, 128))

    def tstep(k, carry):
        a, _ = carry
        mm = jnp.max(a, axis=-1, keepdims=True)
        return jnp.where(a == mm, _NEG_INF, a), mm

    _, th = lax.fori_loop(0, _TOP_K, tstep, (acc, acc0[:, 0:1]))
    th = th - jnp.abs(th) * np.float32(3e-6) - np.float32(1e-33)
    th_ref[...] = jnp.broadcast_to(th, (_RB, 16))


def _sc_body(x_hbm, th_hbm, cv_hbm, ci_hbm, cnt_hbm,
             buf0, buf1, cv_v, ci_v, th_v, cnt_v, cnt_smem, sem0, sem1):
    wid = lax.axis_index("s") * 2 + lax.axis_index("c")
    lane = lax.iota(jnp.int32, 16)
    row0 = wid * _ROWS_PER_W
    pltpu.sync_copy(th_hbm.at[pl.ds(row0 * 16, _ROWS_PER_W * 16)], th_v)

    bufs = (buf0, buf1)
    sems = (sem0, sem1)
    sched = [(rr, c) for rr in range(_ROWS_PER_W) for c in range(_NCH)]

    def issue(t):
        rr, c = sched[t]
        b = t % 2
        return pltpu.async_copy(
            x_hbm.at[pl.ds((row0 + rr) * _V + c * _CH, _CH)], bufs[b], sems[b])

    handle = issue(0)
    for t, (rr, c) in enumerate(sched):
        nxt = issue(t + 1) if t + 1 < len(sched) else None
        handle.wait()
        buf = bufs[t % 2]
        th = th_v[pl.ds(rr * 16, 16)]
        if c == 0:
            cnt_smem[0] = jnp.int32(0)

        def flush(jp, pcv_p, vs_p, th=th, base_c=c * _CH):
            pcg = pcv_p[0]

            @pl.when(pcg > 0)
            def _slow():
                for i in range(_G):
                    m_i = vs_p[i] >= th
                    pci = plsc.all_reduce_population_count(m_i)[0]

                    def _append(i=i, m_i=m_i, pci=pci):
                        off = cnt_smem[0]
                        soff = jnp.minimum(off, jnp.int32(_CAP))
                        kv = jnp.where(m_i, vs_p[i] / _TEMP, _NEG_INF)
                        iv = jnp.where(
                            m_i,
                            base_c + jp * (16 * _G) + np.int32(i * 16) + lane,
                            jnp.int32(0))
                        sk, sv = plsc.sort_key_val(kv, iv, descending=True)
                        cv_v[pl.ds(soff, 16)] = sk
                        ci_v[pl.ds(soff, 16)] = sv
                        cnt_smem[0] = off + pci

                    pl.when(pci > 0)(_append)

        def group(j, carry, buf=buf, th=th):
            pcv_p, vs_p = carry
            vs = [buf[pl.ds(j * (16 * _G) + i * 16, 16)] for i in range(_G)]
            ms = [v >= th for v in vs]
            orm = ms[0]
            for i in range(1, _G):
                orm = orm | ms[i]
            pcv = plsc.all_reduce_population_count(orm)
            flush(j - 1, pcv_p, vs_p)
            return pcv, tuple(vs)

        zv16 = jnp.zeros((16,), jnp.float32)
        pcv_last, vs_last = lax.fori_loop(
            0, _NGRP, group,
            (jnp.zeros((16,), jnp.int32), (zv16,) * _G))
        flush(_NGRP - 1, pcv_last, vs_last)

        if c == _NCH - 1:
            row = row0 + rr
            off = jnp.minimum(cnt_smem[0], jnp.int32(_CAP))
            cnt_v[...] = jnp.full((16,), off, jnp.int32)
            pltpu.sync_copy(cv_v.at[pl.ds(0, _CAP)],
                            cv_hbm.at[pl.ds(row * _CAP, _CAP)])
            pltpu.sync_copy(ci_v.at[pl.ds(0, _CAP)],
                            ci_hbm.at[pl.ds(row * _CAP, _CAP)])
            pltpu.sync_copy(cnt_v, cnt_hbm.at[pl.ds(row * 16, 16)])
        handle = nxt


def _threefry_bits(x1):
    ks = (_KS0, _KS1, _KS2)
    x0 = jnp.zeros_like(x1) + ks[0]
    x1 = x1 + ks[1]

    def rotl(v, r):
        return lax.shift_left(v, np.int32(r)) | lax.shift_right_logical(v, np.int32(32 - r))

    for g in range(5):
        for j in range(4):
            r = _ROTS[g * 4 + j]
            x0 = x0 + x1
            x1 = rotl(x1, r)
            x1 = x1 ^ x0
        x0 = x0 + ks[(g + 1) % 3]
        x1 = x1 + ks[(g + 2) % 3] + np.int32(g + 1)
    return x0 ^ x1


def _fin_body(cv_ref, ci_ref, cnt_ref, z_ref, out_ref):
    cv = cv_ref[...]
    ci = ci_ref[...]
    n = cnt_ref[:, 0:1]
    lane_c = lax.broadcasted_iota(jnp.int32, (_R, _CAP), 1)
    cvm = jnp.where(lane_c < n, cv, _NEG_INF)
    kcol = lax.broadcasted_iota(jnp.int32, (_R, 128), 1)

    def estep(k, carry):
        cva, tv, tix = carry
        mm = jnp.max(cva, axis=-1, keepdims=True)
        tk = jnp.min(jnp.where(cva == mm, ci, jnp.int32(2 ** 30)),
                     axis=-1, keepdims=True)
        kmask = kcol == k
        tv = jnp.where(kmask, mm, tv)
        tix = jnp.where(kmask, tk, tix)
        cva = jnp.where(ci == tk, _NEG_INF, cva)
        return cva, tv, tix

    tv0 = jnp.full((_R, 128), _NEG_INF, jnp.float32)
    ti0 = jnp.zeros((_R, 128), jnp.int32)
    _, v, ti = lax.fori_loop(0, _TOP_K, estep, (cvm, tv0, ti0))

    valid = kcol < _TOP_K
    m = v[:, 0:1]
    ex = jnp.where(valid, jnp.exp(v - m), np.float32(0.0))
    zv = z_ref[:, 0:1]
    p = ex / zv
    ri = lax.broadcasted_iota(jnp.int32, (128, 128), 0)
    cicol = lax.broadcasted_iota(jnp.int32, (128, 128), 1)
    ltri = (ri < cicol).astype(jnp.float32)
    cumprev = lax.dot_general(p, ltri, (((1,), (0,)), ((), ())),
                              precision=lax.Precision.HIGHEST,
                              preferred_element_type=jnp.float32)
    keep = ((cumprev <= _TOP_P) | (kcol == 0)) & valid
    s = jnp.sum(jnp.where(keep, ex, np.float32(0.0)), axis=-1, keepdims=True)
    logp = jnp.log(ex / s)
    row = lax.broadcasted_iota(jnp.int32, (_R, 128), 0)
    flat = row * np.int32(_V) + ti
    bits = _threefry_bits(flat)
    fbits = lax.shift_right_logical(bits, np.int32(9)) | np.int32(0x3F800000)
    frac = lax.bitcast_convert_type(fbits, jnp.float32) - np.float32(1.0)
    u = jnp.maximum(_TINY, frac * (np.float32(1.0) - _TINY) + _TINY)
    g = -jnp.log(-jnp.log(u))
    score = jnp.where(keep, logp + g, _NEG_INF)
    best = jnp.max(score, axis=-1, keepdims=True)
    tok = jnp.min(jnp.where(score == best, ti, jnp.int32(2 ** 30)),
                  axis=-1, keepdims=True)
    out_ref[...] = jnp.broadcast_to(tok, (_R, 128))


_sc_compact = functools.partial(
    pl.kernel,
    out_type=[
        jax.ShapeDtypeStruct((_R * _CAP,), jnp.float32),
        jax.ShapeDtypeStruct((_R * _CAP,), jnp.int32),
        jax.ShapeDtypeStruct((_R * 16,), jnp.int32),
    ],
    mesh=plsc.VectorSubcoreMesh(core_axis_name="c", subcore_axis_name="s",
                                num_cores=2, num_subcores=16),
    scratch_types=[
        pltpu.VMEM((_CH,), jnp.float32),
        pltpu.VMEM((_CH,), jnp.float32),
        pltpu.VMEM((_CAP + 16,), jnp.float32),
        pltpu.VMEM((_CAP + 16,), jnp.int32),
        pltpu.VMEM((_ROWS_PER_W * 16,), jnp.float32),
        pltpu.VMEM((16,), jnp.int32),
        pltpu.SMEM((1,), jnp.int32),
        pltpu.SemaphoreType.DMA,
        pltpu.SemaphoreType.DMA,
    ],
    compiler_params=pltpu.CompilerParams(needs_layout_passes=False),
)(_sc_body)


@jax.jit
def kernel(input_ids, logits):
    del input_ids
    z, th = pl.pallas_call(
        _pre_body,
        grid=(_R // _RB,),
        in_specs=[pl.BlockSpec((_RB, _V), lambda i: (i, 0))],
        out_specs=[
            pl.BlockSpec((_RB, 128), lambda i: (i, 0)),
            pl.BlockSpec((_RB, 16), lambda i: (i, 0)),
        ],
        out_shape=[
            jax.ShapeDtypeStruct((_R, 128), jnp.float32),
            jax.ShapeDtypeStruct((_R, 16), jnp.float32),
        ],
    )(logits)

    cvf, cif, cntf = _sc_compact(logits.reshape(-1), th.reshape(-1))

    out = pl.pallas_call(
        _fin_body,
        out_shape=jax.ShapeDtypeStruct((_R, 128), jnp.int32),
    )(cvf.reshape(_R, _CAP), cif.reshape(_R, _CAP),
      cntf.reshape(_R, 16), z)
    return out[:, 0]
